# Initial kernel scaffold; baseline (speedup 1.0000x reference)
import functools
import jax
import jax.numpy as jnp
from jax import lax
from jax.experimental import pallas as pl
from jax.experimental.pallas import tpu as pltpu
from jax.experimental.pallas import tpu_sc as plsc

N = 10000
E = 320000
G = 64
V = 16
D = 128
H = 256

NPAD = 10240
NPT = NPAD // 16
EROWS = 2512
RPT = EROWS // 16
EPAD = EROWS * 128 - E
CFLAT = NPAD * V

_mesh = plsc.VectorSubcoreMesh(core_axis_name="c", subcore_axis_name="s")


def _rsqrt_sc(d):
    i = lax.bitcast_convert_type(d, jnp.int32)
    i = jnp.int32(0x5F3759DF) - lax.shift_right_logical(i, 1)
    y = lax.bitcast_convert_type(i, jnp.float32)
    for _ in range(3):
        y = y * (1.5 - 0.5 * d * y * y)
    return y


@functools.partial(
    pl.kernel,
    out_type=(
        jax.ShapeDtypeStruct((2, CFLAT), jnp.float32),
        jax.ShapeDtypeStruct((2, NPAD), jnp.float32),
    ),
    mesh=_mesh,
    scratch_types=[
        pltpu.MemorySpace.VMEM_SHARED((CFLAT,), jnp.float32),
        pltpu.MemorySpace.VMEM_SHARED((NPAD,), jnp.float32),
        pltpu.MemorySpace.VMEM_SHARED((NPAD,), jnp.float32),
        pltpu.VMEM((CFLAT // 16,), jnp.float32),
        pltpu.VMEM((NPAD,), jnp.int32),
        pltpu.VMEM((NPAD,), jnp.float32),
        pltpu.VMEM((NPT,), jnp.float32),
        pltpu.VMEM((NPT,), jnp.float32),
        pltpu.VMEM((2, RPT, 128), jnp.int32),
        pltpu.VMEM((128,), jnp.int32),
        pltpu.VMEM((128,), jnp.float32),
        pltpu.VMEM((128,), jnp.float32),
    ],
)
def _sc_deg_c(edges, x_hbm, c_out, dinv_out, c_spm, deg_spm, dinv_spm,
              zero_v, x_v, dinv_v, deg_sl, dinv_sl, edg, idx_v, upd_v, ones_v):
    c = lax.axis_index("c")
    s = lax.axis_index("s")
    nz = CFLAT // 16

    def fill_zero(i, _):
        zero_v[pl.ds(i * 16, 16)] = jnp.zeros((16,), jnp.float32)
        return 0
    lax.fori_loop(0, nz // 16, fill_zero, 0)
    for i in range(8):
        ones_v[pl.ds(i * 16, 16)] = jnp.ones((16,), jnp.float32)

    pltpu.sync_copy(zero_v, c_spm.at[pl.ds(s * nz, nz)])
    pltpu.sync_copy(zero_v.at[pl.ds(0, NPT)], deg_spm.at[pl.ds(s * NPT, NPT)])

    pltpu.sync_copy(edges.at[c, :, pl.ds(s * RPT, RPT)], edg)
    pltpu.sync_copy(x_hbm, x_v)
    plsc.subcore_barrier()

    def deg_row(j, _):
        pltpu.sync_copy(ones_v, deg_spm.at[edg.at[1, j]], add=True)
        return 0
    lax.fori_loop(0, RPT, deg_row, 0)
    plsc.subcore_barrier()

    pltpu.sync_copy(deg_spm.at[pl.ds(s * NPT, NPT)], deg_sl)

    def dinv_blk(i, _):
        d = deg_sl[pl.ds(i * 16, 16)] + 1.0
        dinv_sl[pl.ds(i * 16, 16)] = _rsqrt_sc(d)
        return 0
    lax.fori_loop(0, NPT // 16, dinv_blk, 0)
    pltpu.sync_copy(dinv_sl, dinv_spm.at[pl.ds(s * NPT, NPT)])
    pltpu.sync_copy(dinv_sl, dinv_out.at[c, pl.ds(s * NPT, NPT)])
    plsc.subcore_barrier()

    pltpu.sync_copy(dinv_spm, dinv_v)

    def c_row(j, _):
        for i in range(8):
            sl = pl.ds(i * 16, 16)
            sv = edg[0, j, sl]
            dv = edg[1, j, sl]
            xv = plsc.load_gather(x_v, [sv])
            wv = plsc.load_gather(dinv_v, [sv])
            idx_v[sl] = dv * 16 + xv
            upd_v[sl] = wv
        pltpu.sync_copy(upd_v, c_spm.at[idx_v], add=True)
        return 0
    lax.fori_loop(0, RPT, c_row, 0)
    plsc.subcore_barrier()

    pltpu.sync_copy(c_spm.at[pl.ds(s * nz, nz)], c_out.at[c, pl.ds(s * nz, nz)])


@functools.partial(
    pl.kernel,
    out_type=jax.ShapeDtypeStruct((2, NPAD, D), jnp.float32),
    mesh=_mesh,
    scratch_types=[
        pltpu.MemorySpace.VMEM_SHARED((NPAD, D), jnp.float32),
        pltpu.VMEM((2, RPT, 128), jnp.int32),
        pltpu.VMEM((2, 128, D), jnp.float32),
        pltpu.VMEM((NPAD // 16 - N // 16, D), jnp.float32),
        pltpu.SemaphoreType.DMA,
    ],
)
def _sc_aggregate(edges, t_hbm, acc_out, acc_spm, edg, rows, zrows, sem):
    c = lax.axis_index("c")
    s = lax.axis_index("s")
    npad_rows = NPAD // 16 - N // 16

    def fill_zero(i, _):
        r = i // 8
        k = lax.rem(i, 8)
        zrows[r, pl.ds(k * 16, 16)] = jnp.zeros((16,), jnp.float32)
        return 0
    lax.fori_loop(0, npad_rows * 8, fill_zero, 0)

    nrp = N // 16
    pltpu.sync_copy(t_hbm.at[c, pl.ds(s * nrp, nrp)],
                    acc_spm.at[pl.ds(s * nrp, nrp)])
    pltpu.sync_copy(zrows, acc_spm.at[pl.ds(N + s * npad_rows, npad_rows)])
    pltpu.sync_copy(edges.at[c, :, pl.ds(s * RPT, RPT)], edg)
    plsc.subcore_barrier()

    pltpu.async_copy(t_hbm.at[c].at[edg.at[0, 0]], rows.at[0], sem)

    def agg_row(j, _):
        jm = lax.rem(j, 2)
        pltpu.make_async_copy(
            t_hbm.at[c].at[edg.at[0, j]], rows.at[jm], sem).wait()

        @pl.when(j + 1 < RPT)
        def _():
            pltpu.async_copy(
                t_hbm.at[c].at[edg.at[0, j + 1]], rows.at[1 - jm], sem)
        pltpu.sync_copy(rows.at[jm], acc_spm.at[edg.at[1, j]], add=True)
        return 0
    lax.fori_loop(0, RPT, agg_row, 0)
    plsc.subcore_barrier()

    nout = NPAD // 16
    pltpu.sync_copy(acc_spm.at[pl.ds(s * nout, nout)],
                    acc_out.at[c, pl.ds(s * nout, nout)])


def _tc1_body(c_ref, dinv_ref, x_ref, emb_ref, w1_ref, b1_ref, w2_ref, t_ref):
    cb = c_ref[0]
    dv = dinv_ref[0, 0, 0]
    xv = x_ref[0, 0]
    bn = cb.shape[0]
    emb_w1 = jnp.dot(emb_ref[...], w1_ref[0], preferred_element_type=jnp.float32)
    oh = (lax.broadcasted_iota(jnp.int32, (bn, V), 1) == xv[:, None])
    ctot = cb * dv[:, None] + oh.astype(jnp.float32) * (dv * dv)[:, None]
    z1 = jnp.dot(ctot, emb_w1,
                 preferred_element_type=jnp.float32) + b1_ref[0, 0][None, :]
    h1 = jnp.where(z1 > 0, z1, jnp.expm1(z1))
    t = jnp.dot(h1, w2_ref[0], preferred_element_type=jnp.float32) * dv[:, None]
    t_ref[0] = t


def _tc1(c3, dinv4, x3, emb, w1s, b1s, w2s):
    bn = 1024
    nb = NPAD // bn
    return pl.pallas_call(
        _tc1_body,
        grid=(2, nb),
        in_specs=[
            pl.BlockSpec((1, bn, V), lambda v, b: (v, b, 0)),
            pl.BlockSpec((1, 1, 1, bn), lambda v, b: (v, b, 0, 0)),
            pl.BlockSpec((1, 1, bn), lambda v, b: (b, 0, 0)),
            pl.BlockSpec((V, D), lambda v, b: (0, 0)),
            pl.BlockSpec((1, D, H), lambda v, b: (v, 0, 0)),
            pl.BlockSpec((1, 1, H), lambda v, b: (v, 0, 0)),
            pl.BlockSpec((1, H, D), lambda v, b: (v, 0, 0)),
        ],
        out_specs=pl.BlockSpec((1, bn, D), lambda v, b: (v, b, 0)),
        out_shape=jax.ShapeDtypeStruct((2, NPAD, D), jnp.float32),
    )(c3, dinv4, x3, emb, w1s, b1s, w2s)


def _tc2_body(acc_ref, dinv_ref, b2_ref, batch_ref, node_ref, graph_ref,
              gacc, gcnt):
    b = pl.program_id(0)
    nb = pl.num_programs(0)
    bn = acc_ref.shape[1]

    @pl.when(b == 0)
    def _():
        gacc[...] = jnp.zeros_like(gacc)
        gcnt[...] = jnp.zeros_like(gcnt)

    hs = []
    for v in range(2):
        z = acc_ref[v] * dinv_ref[v, 0, 0][:, None] + b2_ref[v][None, :]
        hs.append(jnp.where(z > 0, z, jnp.expm1(z)))
    fused = jnp.concatenate(hs, axis=1)
    node_ref[...] = fused

    row = b * bn + lax.broadcasted_iota(jnp.int32, (1, bn), 1)[0]
    valid = row < N
    bv = batch_ref[0, 0]
    pt = (lax.broadcasted_iota(jnp.int32, (G, bn), 0) == bv[None, :]) \
        & valid[None, :]
    ptf = pt.astype(jnp.float32)
    g_new = gacc[...] + jnp.dot(ptf, fused, preferred_element_type=jnp.float32)
    c_new = gcnt[...] + jnp.dot(ptf, jnp.ones((bn, D), jnp.float32),
                                preferred_element_type=jnp.float32)
    gacc[...] = g_new
    gcnt[...] = c_new

    @pl.when(b == nb - 1)
    def _():
        graph_ref[...] = g_new / jnp.maximum(c_new[:, 0:1], 1.0)


def _tc2(acc, dinv4, b2s, batch3):
    bn = 1024
    nb = NPAD // bn
    return pl.pallas_call(
        _tc2_body,
        grid=(nb,),
        in_specs=[
            pl.BlockSpec((2, bn, D), lambda b: (0, b, 0)),
            pl.BlockSpec((2, 1, 1, bn), lambda b: (0, b, 0, 0)),
            pl.BlockSpec((2, D), lambda b: (0, 0)),
            pl.BlockSpec((1, 1, bn), lambda b: (b, 0, 0)),
        ],
        out_specs=[
            pl.BlockSpec((bn, 2 * D), lambda b: (b, 0)),
            pl.BlockSpec((G, 2 * D), lambda b: (0, 0)),
        ],
        out_shape=[
            jax.ShapeDtypeStruct((N, 2 * D), jnp.float32),
            jax.ShapeDtypeStruct((G, 2 * D), jnp.float32),
        ],
        scratch_shapes=[
            pltpu.VMEM((G, 2 * D), jnp.float32),
            pltpu.VMEM((G, D), jnp.float32),
        ],
    )(acc, dinv4, b2s, batch3)


def kernel(x, edge_index_seq, edge_index_pair, batch, emb,
           Ws1, bs1, Ws2, bs2, Wp1, bp1, Wp2, bp2):
    i32 = jnp.int32
    ei = jnp.stack([edge_index_seq.astype(i32), edge_index_pair.astype(i32)])
    pad_src = jnp.broadcast_to((jnp.arange(EPAD, dtype=i32) % 128)[None, :],
                               (2, EPAD))
    pad_dst = jnp.full((2, EPAD), N, i32)
    src = jnp.concatenate([ei[:, 0, :], pad_src], axis=1)
    dst = jnp.concatenate([ei[:, 1, :], pad_dst], axis=1)
    edges = jnp.stack([src, dst], axis=1).reshape(2, 2, EROWS, 128)

    x_p = jnp.concatenate([x.astype(i32), jnp.zeros((NPAD - N,), i32)])
    batch_p = jnp.concatenate([batch.astype(i32), jnp.zeros((NPAD - N,), i32)])

    w1s = jnp.stack([Ws1, Wp1])
    b1s = jnp.stack([bs1, bp1]).reshape(2, 1, H)
    w2s = jnp.stack([Ws2, Wp2])
    b2s = jnp.stack([bs2, bp2])

    c_flat, dinv = _sc_deg_c(edges, x_p)
    c3 = c_flat.reshape(2, NPAD, V)
    dinv4 = dinv.reshape(2, NPAD // 1024, 1, 1024)
    x3 = x_p.reshape(NPAD // 1024, 1, 1024)
    batch3 = batch_p.reshape(NPAD // 1024, 1, 1024)

    t = _tc1(c3, dinv4, x3, emb, w1s, b1s, w2s)
    acc = _sc_aggregate(edges, t)
    fused_node, fused_graph = _tc2(acc, dinv4, b2s, batch3)
    return (fused_graph, fused_node)


# speedup vs baseline: 40.1778x; 40.1778x over previous
import functools
import jax
import jax.numpy as jnp
from jax import lax
from jax.experimental import pallas as pl
from jax.experimental.pallas import tpu as pltpu
from jax.experimental.pallas import tpu_sc as plsc

N = 10000
E = 320000
G = 64
V = 16
D = 128
H = 256

NPAD = 10240
NPT = NPAD // 16
EROWS = 2560
RPT = EROWS // 16
ECH = 16
NCH = RPT // ECH
EPAD = EROWS * 128 - E
CFLAT = NPAD * V

_sc_kernel_cache = {}


def _sc_mesh():
    return plsc.VectorSubcoreMesh(core_axis_name="c", subcore_axis_name="s",
                                  num_cores=2, num_subcores=16)


def _rsqrt_sc(d):
    i = lax.bitcast_convert_type(d, jnp.int32)
    i = jnp.int32(0x5F3759DF) - lax.shift_right_logical(i, 1)
    y = lax.bitcast_convert_type(i, jnp.float32)
    for _ in range(3):
        y = y * (1.5 - 0.5 * d * y * y)
    return y


def _sc_deg_c(edges, edges_flat, x_p):
    if "deg_c" not in _sc_kernel_cache:
        _sc_kernel_cache["deg_c"] = functools.partial(
            pl.kernel,
            out_type=(
                jax.ShapeDtypeStruct((2, CFLAT), jnp.float32),
                jax.ShapeDtypeStruct((2, NPAD), jnp.float32),
            ),
            mesh=_sc_mesh(),
            compiler_params=pltpu.CompilerParams(needs_layout_passes=False),
            scratch_types=[
                pltpu.MemorySpace.VMEM_SHARED((CFLAT,), jnp.float32),
                pltpu.MemorySpace.VMEM_SHARED((NPAD,), jnp.float32),
                pltpu.MemorySpace.VMEM_SHARED((NPAD,), jnp.float32),
                pltpu.VMEM((CFLAT // 16,), jnp.float32),
                pltpu.VMEM((NPAD,), jnp.int32),
                pltpu.VMEM((NPAD,), jnp.float32),
                pltpu.VMEM((NPT,), jnp.float32),
                pltpu.VMEM((NPT,), jnp.float32),
                pltpu.VMEM((RPT, 128), jnp.int32),
                pltpu.VMEM((RPT * 128,), jnp.int32),
                pltpu.VMEM((RPT * 128,), jnp.int32),
                pltpu.VMEM((128,), jnp.int32),
                pltpu.VMEM((128,), jnp.float32),
                pltpu.VMEM((128,), jnp.float32),
            ],
        )(_sc_deg_c_body)
    return _sc_kernel_cache["deg_c"](edges, edges_flat, x_p)


def _sc_deg_c_body(edges, edges_flat, x_hbm, c_out, dinv_out,
                   c_spm, deg_spm, dinv_spm,
                   zero_v, x_v, dinv_v, deg_sl, dinv_sl,
                   edgd, srcf, dstf, idx_v, upd_v, ones_v):
    c = lax.axis_index("c")
    s = lax.axis_index("s")
    nz = CFLAT // 16

    def fill_zero(i, _):
        zero_v[pl.ds(i * 16, 16)] = jnp.zeros((16,), jnp.float32)
        return 0
    lax.fori_loop(0, nz // 16, fill_zero, 0)
    for i in range(8):
        ones_v[pl.ds(i * 16, 16)] = jnp.ones((16,), jnp.float32)

    pltpu.sync_copy(zero_v, c_spm.at[pl.ds(s * nz, nz)])
    pltpu.sync_copy(zero_v.at[pl.ds(0, NPT)], deg_spm.at[pl.ds(s * NPT, NPT)])

    pltpu.sync_copy(edges.at[c, 1, s], edgd)
    pltpu.sync_copy(edges_flat.at[c, 0, s], srcf)
    pltpu.sync_copy(edges_flat.at[c, 1, s], dstf)
    pltpu.sync_copy(x_hbm, x_v)
    plsc.subcore_barrier()

    def deg_row(j, _):
        pltpu.sync_copy(ones_v, deg_spm.at[edgd.at[j]], add=True)
        return 0
    lax.fori_loop(0, RPT, deg_row, 0)
    plsc.subcore_barrier()

    pltpu.sync_copy(deg_spm.at[pl.ds(s * NPT, NPT)], deg_sl)

    def dinv_blk(i, _):
        d = deg_sl[pl.ds(i * 16, 16)] + 1.0
        dinv_sl[pl.ds(i * 16, 16)] = _rsqrt_sc(d)
        return 0
    lax.fori_loop(0, NPT // 16, dinv_blk, 0)
    pltpu.sync_copy(dinv_sl, dinv_spm.at[pl.ds(s * NPT, NPT)])
    pltpu.sync_copy(dinv_sl, dinv_out.at[c, pl.ds(s * NPT, NPT)])
    plsc.subcore_barrier()

    pltpu.sync_copy(dinv_spm, dinv_v)

    def c_row(j, _):
        for i in range(8):
            sl = pl.ds(i * 16, 16)
            fl = pl.ds(j * 128 + i * 16, 16)
            sv = srcf[fl]
            dv = dstf[fl]
            xv = plsc.load_gather(x_v, [sv])
            wv = plsc.load_gather(dinv_v, [sv])
            idx_v[sl] = dv * 16 + xv
            upd_v[sl] = wv
        pltpu.sync_copy(upd_v, c_spm.at[idx_v], add=True)
        return 0
    lax.fori_loop(0, RPT, c_row, 0)
    plsc.subcore_barrier()

    pltpu.sync_copy(c_spm.at[pl.ds(s * nz, nz)], c_out.at[c, pl.ds(s * nz, nz)])


def _sc_aggregate(edges, t):
    if "agg" not in _sc_kernel_cache:
        _sc_kernel_cache["agg"] = functools.partial(
            pl.kernel,
            out_type=jax.ShapeDtypeStruct((2, NPAD, D), jnp.float32),
            mesh=_sc_mesh(),
            compiler_params=pltpu.CompilerParams(needs_layout_passes=False),
            scratch_types=[
                pltpu.MemorySpace.VMEM_SHARED((NPAD, D), jnp.float32),
                pltpu.VMEM((2, ECH, 128), jnp.int32),
                pltpu.VMEM((2, 128, D), jnp.float32),
                pltpu.SemaphoreType.DMA,
            ],
        )(_sc_aggregate_body)
    return _sc_kernel_cache["agg"](edges, t)


def _sc_aggregate_body(edges, t_hbm, acc_out, acc_spm, edg, rows, sem):
    c = lax.axis_index("c")
    s = lax.axis_index("s")

    nrp = NPAD // 16
    pltpu.sync_copy(t_hbm.at[c, pl.ds(s * nrp, nrp)],
                    acc_spm.at[pl.ds(s * nrp, nrp)])
    plsc.subcore_barrier()

    def chunk(k, _):
        pltpu.sync_copy(edges.at[c, :, s, pl.ds(k * ECH, ECH)], edg)
        pltpu.async_copy(t_hbm.at[c].at[edg.at[0, 0]], rows.at[0], sem)

        def agg_row(j, _):
            jm = lax.rem(j, 2)
            pltpu.make_async_copy(
                t_hbm.at[c].at[edg.at[0, j]], rows.at[jm], sem).wait()

            @pl.when(j + 1 < ECH)
            def _():
                pltpu.async_copy(
                    t_hbm.at[c].at[edg.at[0, j + 1]], rows.at[1 - jm], sem)
            pltpu.sync_copy(rows.at[jm], acc_spm.at[edg.at[1, j]], add=True)
            return 0
        lax.fori_loop(0, ECH, agg_row, 0)
        return 0
    lax.fori_loop(0, NCH, chunk, 0)
    plsc.subcore_barrier()

    nout = NPAD // 16
    pltpu.sync_copy(acc_spm.at[pl.ds(s * nout, nout)],
                    acc_out.at[c, pl.ds(s * nout, nout)])


def _tc1_body(c_ref, dinv_ref, x_ref, emb_ref, w1_ref, b1_ref, w2_ref, t_ref):
    cb = c_ref[0]
    dv = dinv_ref[0, 0, 0]
    xv = x_ref[0, 0]
    bn = cb.shape[0]
    emb_w1 = jnp.dot(emb_ref[...], w1_ref[0], preferred_element_type=jnp.float32)
    oh = (lax.broadcasted_iota(jnp.int32, (bn, V), 1) == xv[:, None])
    ctot = cb * dv[:, None] + oh.astype(jnp.float32) * (dv * dv)[:, None]
    z1 = jnp.dot(ctot, emb_w1,
                 preferred_element_type=jnp.float32) + b1_ref[0, 0][None, :]
    h1 = jnp.where(z1 > 0, z1, jnp.exp(z1) - 1.0)
    t = jnp.dot(h1, w2_ref[0], preferred_element_type=jnp.float32) * dv[:, None]
    t_ref[0] = t


def _tc1(c3, dinv4, x3, emb, w1s, b1s, w2s):
    bn = 1024
    nb = NPAD // bn
    return pl.pallas_call(
        _tc1_body,
        grid=(2, nb),
        in_specs=[
            pl.BlockSpec((1, bn, V), lambda v, b: (v, b, 0)),
            pl.BlockSpec((1, 1, 1, bn), lambda v, b: (v, b, 0, 0)),
            pl.BlockSpec((1, 1, bn), lambda v, b: (b, 0, 0)),
            pl.BlockSpec((V, D), lambda v, b: (0, 0)),
            pl.BlockSpec((1, D, H), lambda v, b: (v, 0, 0)),
            pl.BlockSpec((1, 1, H), lambda v, b: (v, 0, 0)),
            pl.BlockSpec((1, H, D), lambda v, b: (v, 0, 0)),
        ],
        out_specs=pl.BlockSpec((1, bn, D), lambda v, b: (v, b, 0)),
        out_shape=jax.ShapeDtypeStruct((2, NPAD, D), jnp.float32),
    )(c3, dinv4, x3, emb, w1s, b1s, w2s)


def _tc2_body(acc_ref, dinv_ref, b2_ref, batch_ref, node_ref, graph_ref,
              gacc, gcnt):
    b = pl.program_id(0)
    nb = pl.num_programs(0)
    bn = acc_ref.shape[1]

    @pl.when(b == 0)
    def _():
        gacc[...] = jnp.zeros_like(gacc)
        gcnt[...] = jnp.zeros_like(gcnt)

    hs = []
    for v in range(2):
        z = acc_ref[v] * dinv_ref[v, 0, 0][:, None] + b2_ref[v][None, :]
        hs.append(jnp.where(z > 0, z, jnp.exp(z) - 1.0))
    fused = jnp.concatenate(hs, axis=1)
    node_ref[...] = fused

    row = b * bn + lax.broadcasted_iota(jnp.int32, (1, bn), 1)[0]
    valid = row < N
    bv = batch_ref[0, 0]
    pt = (lax.broadcasted_iota(jnp.int32, (G, bn), 0) == bv[None, :]) \
        & valid[None, :]
    ptf = pt.astype(jnp.float32)
    g_new = gacc[...] + jnp.dot(ptf, fused, preferred_element_type=jnp.float32)
    c_new = gcnt[...] + jnp.dot(ptf, jnp.ones((bn, D), jnp.float32),
                                preferred_element_type=jnp.float32)
    gacc[...] = g_new
    gcnt[...] = c_new

    @pl.when(b == nb - 1)
    def _():
        graph_ref[...] = g_new / jnp.maximum(c_new[:, 0:1], 1.0)


def _tc2(acc, dinv4, b2s, batch3):
    bn = 1024
    nb = NPAD // bn
    return pl.pallas_call(
        _tc2_body,
        grid=(nb,),
        in_specs=[
            pl.BlockSpec((2, bn, D), lambda b: (0, b, 0)),
            pl.BlockSpec((2, 1, 1, bn), lambda b: (0, b, 0, 0)),
            pl.BlockSpec((2, D), lambda b: (0, 0)),
            pl.BlockSpec((1, 1, bn), lambda b: (b, 0, 0)),
        ],
        out_specs=[
            pl.BlockSpec((bn, 2 * D), lambda b: (b, 0)),
            pl.BlockSpec((G, 2 * D), lambda b: (0, 0)),
        ],
        out_shape=[
            jax.ShapeDtypeStruct((N, 2 * D), jnp.float32),
            jax.ShapeDtypeStruct((G, 2 * D), jnp.float32),
        ],
        scratch_shapes=[
            pltpu.VMEM((G, 2 * D), jnp.float32),
            pltpu.VMEM((G, D), jnp.float32),
        ],
    )(acc, dinv4, b2s, batch3)


def kernel(x, edge_index_seq, edge_index_pair, batch, emb,
           Ws1, bs1, Ws2, bs2, Wp1, bp1, Wp2, bp2):
    i32 = jnp.int32
    ei = jnp.stack([edge_index_seq.astype(i32), edge_index_pair.astype(i32)])
    pad_src = jnp.broadcast_to((jnp.arange(EPAD, dtype=i32) % 128)[None, :],
                               (2, EPAD))
    pad_dst = jnp.full((2, EPAD), N, i32)
    src = jnp.concatenate([ei[:, 0, :], pad_src], axis=1)
    dst = jnp.concatenate([ei[:, 1, :], pad_dst], axis=1)
    edges = jnp.stack([src, dst], axis=1).reshape(2, 2, 16, RPT, 128)
    edges_flat = edges.reshape(2, 2, 16, RPT * 128)

    x_p = jnp.concatenate([x.astype(i32), jnp.zeros((NPAD - N,), i32)])
    batch_p = jnp.concatenate([batch.astype(i32), jnp.zeros((NPAD - N,), i32)])

    w1s = jnp.stack([Ws1, Wp1])
    b1s = jnp.stack([bs1, bp1]).reshape(2, 1, H)
    w2s = jnp.stack([Ws2, Wp2])
    b2s = jnp.stack([bs2, bp2])

    c_flat, dinv = _sc_deg_c(edges, edges_flat, x_p)
    c3 = c_flat.reshape(2, NPAD, V)
    dinv4 = dinv.reshape(2, NPAD // 1024, 1, 1024)
    x3 = x_p.reshape(NPAD // 1024, 1, 1024)
    batch3 = batch_p.reshape(NPAD // 1024, 1, 1024)

    t = _tc1(c3, dinv4, x3, emb, w1s, b1s, w2s)
    acc = _sc_aggregate(edges, t)
    fused_node, fused_graph = _tc2(acc, dinv4, b2s, batch3)
    return (fused_graph, fused_node)


# speedup vs baseline: 40.8700x; 1.0172x over previous
import functools
import jax
import jax.numpy as jnp
from jax import lax
from jax.experimental import pallas as pl
from jax.experimental.pallas import tpu as pltpu
from jax.experimental.pallas import tpu_sc as plsc

N = 10000
E = 320000
G = 64
V = 16
D = 128
H = 256

NPAD = 10240
NPT = NPAD // 16
EROWS = 2560
RPT = EROWS // 16
ECH = 16
NCH = RPT // ECH
EPAD = EROWS * 128 - E
CFLAT = NPAD * V

_sc_kernel_cache = {}


def _sc_mesh():
    return plsc.VectorSubcoreMesh(core_axis_name="c", subcore_axis_name="s",
                                  num_cores=2, num_subcores=16)


def _rsqrt_sc(d):
    i = lax.bitcast_convert_type(d, jnp.int32)
    i = jnp.int32(0x5F3759DF) - lax.shift_right_logical(i, 1)
    y = lax.bitcast_convert_type(i, jnp.float32)
    for _ in range(3):
        y = y * (1.5 - 0.5 * d * y * y)
    return y


def _sc_deg_c(edges, edges_flat, x_p):
    if "deg_c" not in _sc_kernel_cache:
        _sc_kernel_cache["deg_c"] = functools.partial(
            pl.kernel,
            out_type=(
                jax.ShapeDtypeStruct((2, CFLAT), jnp.float32),
                jax.ShapeDtypeStruct((2, NPAD), jnp.float32),
            ),
            mesh=_sc_mesh(),
            compiler_params=pltpu.CompilerParams(needs_layout_passes=False),
            scratch_types=[
                pltpu.MemorySpace.VMEM_SHARED((CFLAT,), jnp.float32),
                pltpu.MemorySpace.VMEM_SHARED((NPAD,), jnp.float32),
                pltpu.MemorySpace.VMEM_SHARED((NPAD,), jnp.float32),
                pltpu.VMEM((CFLAT // 16,), jnp.float32),
                pltpu.VMEM((NPAD,), jnp.int32),
                pltpu.VMEM((NPAD,), jnp.float32),
                pltpu.VMEM((NPT,), jnp.float32),
                pltpu.VMEM((NPT,), jnp.float32),
                pltpu.VMEM((RPT, 128), jnp.int32),
                pltpu.VMEM((RPT * 128,), jnp.int32),
                pltpu.VMEM((RPT * 128,), jnp.int32),
                pltpu.VMEM((128,), jnp.int32),
                pltpu.VMEM((128,), jnp.float32),
                pltpu.VMEM((128,), jnp.float32),
            ],
        )(_sc_deg_c_body)
    return _sc_kernel_cache["deg_c"](edges, edges_flat, x_p)


def _sc_deg_c_body(edges, edges_flat, x_hbm, c_out, dinv_out,
                   c_spm, deg_spm, dinv_spm,
                   zero_v, x_v, dinv_v, deg_sl, dinv_sl,
                   edgd, srcf, dstf, idx_v, upd_v, ones_v):
    c = lax.axis_index("c")
    s = lax.axis_index("s")
    nz = CFLAT // 16

    def fill_zero(i, _):
        zero_v[pl.ds(i * 16, 16)] = jnp.zeros((16,), jnp.float32)
        return 0
    lax.fori_loop(0, nz // 16, fill_zero, 0)
    for i in range(8):
        ones_v[pl.ds(i * 16, 16)] = jnp.ones((16,), jnp.float32)

    pltpu.sync_copy(zero_v, c_spm.at[pl.ds(s * nz, nz)])
    pltpu.sync_copy(zero_v.at[pl.ds(0, NPT)], deg_spm.at[pl.ds(s * NPT, NPT)])

    pltpu.sync_copy(edges.at[c, 1, s], edgd)
    pltpu.sync_copy(edges_flat.at[c, 0, s], srcf)
    pltpu.sync_copy(edges_flat.at[c, 1, s], dstf)
    pltpu.sync_copy(x_hbm, x_v)
    plsc.subcore_barrier()

    def deg_row(j, _):
        pltpu.sync_copy(ones_v, deg_spm.at[edgd.at[j]], add=True)
        return 0
    lax.fori_loop(0, RPT, deg_row, 0)
    plsc.subcore_barrier()

    pltpu.sync_copy(deg_spm.at[pl.ds(s * NPT, NPT)], deg_sl)

    def dinv_blk(i, _):
        d = deg_sl[pl.ds(i * 16, 16)] + 1.0
        dinv_sl[pl.ds(i * 16, 16)] = _rsqrt_sc(d)
        return 0
    lax.fori_loop(0, NPT // 16, dinv_blk, 0)
    pltpu.sync_copy(dinv_sl, dinv_spm.at[pl.ds(s * NPT, NPT)])
    pltpu.sync_copy(dinv_sl, dinv_out.at[c, pl.ds(s * NPT, NPT)])
    plsc.subcore_barrier()

    pltpu.sync_copy(dinv_spm, dinv_v)

    def c_row(j, _):
        for i in range(8):
            sl = pl.ds(i * 16, 16)
            fl = pl.ds(j * 128 + i * 16, 16)
            sv = srcf[fl]
            dv = dstf[fl]
            xv = plsc.load_gather(x_v, [sv])
            wv = plsc.load_gather(dinv_v, [sv])
            idx_v[sl] = dv * 16 + xv
            upd_v[sl] = wv
        pltpu.sync_copy(upd_v, c_spm.at[idx_v], add=True)
        return 0
    lax.fori_loop(0, RPT, c_row, 0)
    plsc.subcore_barrier()

    pltpu.sync_copy(c_spm.at[pl.ds(s * nz, nz)], c_out.at[c, pl.ds(s * nz, nz)])


def _sc_aggregate(edges, t):
    if "agg" not in _sc_kernel_cache:
        _sc_kernel_cache["agg"] = functools.partial(
            pl.kernel,
            out_type=jax.ShapeDtypeStruct((2, NPAD, D), jnp.float32),
            mesh=_sc_mesh(),
            compiler_params=pltpu.CompilerParams(needs_layout_passes=False),
            scratch_types=[
                pltpu.MemorySpace.VMEM_SHARED((NPAD, D), jnp.float32),
                pltpu.VMEM((2, 2, ECH, 128), jnp.int32),
                pltpu.VMEM((2, 128, D), jnp.float32),
                pltpu.SemaphoreType.DMA,
                pltpu.SemaphoreType.DMA,
                pltpu.SemaphoreType.DMA,
            ],
        )(_sc_aggregate_body)
    return _sc_kernel_cache["agg"](edges, t)


def _sc_aggregate_body(edges, t_hbm, acc_out, acc_spm, edg, rows,
                       seme, semg, sems):
    c = lax.axis_index("c")
    s = lax.axis_index("s")

    nrp = NPAD // 16
    pltpu.sync_copy(t_hbm.at[c, pl.ds(s * nrp, nrp)],
                    acc_spm.at[pl.ds(s * nrp, nrp)])
    plsc.subcore_barrier()

    def echunk(k, km):
        return pltpu.make_async_copy(
            edges.at[c, :, s, pl.ds(k * ECH, ECH)], edg.at[km], seme)

    def gat(j, km, jj, jm):
        return pltpu.make_async_copy(
            t_hbm.at[c].at[edg.at[km, 0, jj]], rows.at[jm], semg)

    def sca(j, km, jj, jm):
        return pltpu.make_async_copy(
            rows.at[jm], acc_spm.at[edg.at[km, 1, jj]], sems)

    echunk(0, 0).start()
    echunk(0, 0).wait()

    @pl.when(NCH > 1)
    def _():
        echunk(1, 1).start()
    pltpu.async_copy(t_hbm.at[c].at[edg.at[0, 0, 0]], rows.at[0], semg)

    def agg_row(j, _):
        km = lax.rem(j // ECH, 2)
        jj = lax.rem(j, ECH)
        jm = lax.rem(j, 2)
        gat(j, km, jj, jm).wait()
        pltpu.async_copy(rows.at[jm], acc_spm.at[edg.at[km, 1, jj]], sems,
                         add=True)

        @pl.when(j >= 1)
        def _():
            sca(j, km, jj, 1 - jm).wait()

        @pl.when((jj == 0) & (j >= ECH) & (j + ECH < RPT))
        def _():
            echunk(j // ECH + 1, lax.rem(j // ECH + 1, 2)).start()

        @pl.when(jnp.logical_and(jj == ECH - 1, j + 1 < RPT))
        def _():
            echunk((j + 1) // ECH, lax.rem((j + 1) // ECH, 2)).wait()

        @pl.when(j + 1 < RPT)
        def _():
            km2 = lax.rem((j + 1) // ECH, 2)
            jj2 = lax.rem(j + 1, ECH)
            pltpu.async_copy(
                t_hbm.at[c].at[edg.at[km2, 0, jj2]], rows.at[1 - jm], semg)
        return 0
    lax.fori_loop(0, RPT, agg_row, 0)
    sca(RPT - 1, 0, 0, lax.rem(RPT - 1, 2)).wait()
    plsc.subcore_barrier()

    nout = NPAD // 16
    pltpu.sync_copy(acc_spm.at[pl.ds(s * nout, nout)],
                    acc_out.at[c, pl.ds(s * nout, nout)])


def _tc1_body(c_ref, dinv_ref, x_ref, emb_ref, w1_ref, b1_ref, w2_ref, t_ref):
    cb = c_ref[0]
    dv = dinv_ref[0, 0, 0]
    xv = x_ref[0, 0]
    bn = cb.shape[0]
    emb_w1 = jnp.dot(emb_ref[...], w1_ref[0], preferred_element_type=jnp.float32)
    oh = (lax.broadcasted_iota(jnp.int32, (bn, V), 1) == xv[:, None])
    ctot = cb * dv[:, None] + oh.astype(jnp.float32) * (dv * dv)[:, None]
    z1 = jnp.dot(ctot, emb_w1,
                 preferred_element_type=jnp.float32) + b1_ref[0, 0][None, :]
    h1 = jnp.where(z1 > 0, z1, jnp.exp(z1) - 1.0)
    t = jnp.dot(h1, w2_ref[0], preferred_element_type=jnp.float32) * dv[:, None]
    t_ref[0] = t


def _tc1(c3, dinv4, x3, emb, w1s, b1s, w2s):
    bn = 1024
    nb = NPAD // bn
    return pl.pallas_call(
        _tc1_body,
        grid=(2, nb),
        in_specs=[
            pl.BlockSpec((1, bn, V), lambda v, b: (v, b, 0)),
            pl.BlockSpec((1, 1, 1, bn), lambda v, b: (v, b, 0, 0)),
            pl.BlockSpec((1, 1, bn), lambda v, b: (b, 0, 0)),
            pl.BlockSpec((V, D), lambda v, b: (0, 0)),
            pl.BlockSpec((1, D, H), lambda v, b: (v, 0, 0)),
            pl.BlockSpec((1, 1, H), lambda v, b: (v, 0, 0)),
            pl.BlockSpec((1, H, D), lambda v, b: (v, 0, 0)),
        ],
        out_specs=pl.BlockSpec((1, bn, D), lambda v, b: (v, b, 0)),
        out_shape=jax.ShapeDtypeStruct((2, NPAD, D), jnp.float32),
    )(c3, dinv4, x3, emb, w1s, b1s, w2s)


def _tc2_body(acc_ref, dinv_ref, b2_ref, batch_ref, node_ref, graph_ref,
              gacc, gcnt):
    b = pl.program_id(0)
    nb = pl.num_programs(0)
    bn = acc_ref.shape[1]

    @pl.when(b == 0)
    def _():
        gacc[...] = jnp.zeros_like(gacc)
        gcnt[...] = jnp.zeros_like(gcnt)

    hs = []
    for v in range(2):
        z = acc_ref[v] * dinv_ref[v, 0, 0][:, None] + b2_ref[v][None, :]
        hs.append(jnp.where(z > 0, z, jnp.exp(z) - 1.0))
    fused = jnp.concatenate(hs, axis=1)
    node_ref[...] = fused

    row = b * bn + lax.broadcasted_iota(jnp.int32, (1, bn), 1)[0]
    valid = row < N
    bv = batch_ref[0, 0]
    pt = (lax.broadcasted_iota(jnp.int32, (G, bn), 0) == bv[None, :]) \
        & valid[None, :]
    ptf = pt.astype(jnp.float32)
    g_new = gacc[...] + jnp.dot(ptf, fused, preferred_element_type=jnp.float32)
    c_new = gcnt[...] + jnp.dot(ptf, jnp.ones((bn, D), jnp.float32),
                                preferred_element_type=jnp.float32)
    gacc[...] = g_new
    gcnt[...] = c_new

    @pl.when(b == nb - 1)
    def _():
        graph_ref[...] = g_new / jnp.maximum(c_new[:, 0:1], 1.0)


def _tc2(acc, dinv4, b2s, batch3):
    bn = 1024
    nb = NPAD // bn
    return pl.pallas_call(
        _tc2_body,
        grid=(nb,),
        in_specs=[
            pl.BlockSpec((2, bn, D), lambda b: (0, b, 0)),
            pl.BlockSpec((2, 1, 1, bn), lambda b: (0, b, 0, 0)),
            pl.BlockSpec((2, D), lambda b: (0, 0)),
            pl.BlockSpec((1, 1, bn), lambda b: (b, 0, 0)),
        ],
        out_specs=[
            pl.BlockSpec((bn, 2 * D), lambda b: (b, 0)),
            pl.BlockSpec((G, 2 * D), lambda b: (0, 0)),
        ],
        out_shape=[
            jax.ShapeDtypeStruct((N, 2 * D), jnp.float32),
            jax.ShapeDtypeStruct((G, 2 * D), jnp.float32),
        ],
        scratch_shapes=[
            pltpu.VMEM((G, 2 * D), jnp.float32),
            pltpu.VMEM((G, D), jnp.float32),
        ],
    )(acc, dinv4, b2s, batch3)


def kernel(x, edge_index_seq, edge_index_pair, batch, emb,
           Ws1, bs1, Ws2, bs2, Wp1, bp1, Wp2, bp2):
    i32 = jnp.int32
    ei = jnp.stack([edge_index_seq.astype(i32), edge_index_pair.astype(i32)])
    pad_src = jnp.broadcast_to((jnp.arange(EPAD, dtype=i32) % 128)[None, :],
                               (2, EPAD))
    pad_dst = jnp.full((2, EPAD), N, i32)
    src = jnp.concatenate([ei[:, 0, :], pad_src], axis=1)
    dst = jnp.concatenate([ei[:, 1, :], pad_dst], axis=1)
    edges = jnp.stack([src, dst], axis=1).reshape(2, 2, 16, RPT, 128)
    edges_flat = edges.reshape(2, 2, 16, RPT * 128)

    x_p = jnp.concatenate([x.astype(i32), jnp.zeros((NPAD - N,), i32)])
    batch_p = jnp.concatenate([batch.astype(i32), jnp.zeros((NPAD - N,), i32)])

    w1s = jnp.stack([Ws1, Wp1])
    b1s = jnp.stack([bs1, bp1]).reshape(2, 1, H)
    w2s = jnp.stack([Ws2, Wp2])
    b2s = jnp.stack([bs2, bp2])

    c_flat, dinv = _sc_deg_c(edges, edges_flat, x_p)
    c3 = c_flat.reshape(2, NPAD, V)
    dinv4 = dinv.reshape(2, NPAD // 1024, 1, 1024)
    x3 = x_p.reshape(NPAD // 1024, 1, 1024)
    batch3 = batch_p.reshape(NPAD // 1024, 1, 1024)

    t = _tc1(c3, dinv4, x3, emb, w1s, b1s, w2s)
    acc = _sc_aggregate(edges, t)
    fused_node, fused_graph = _tc2(acc, dinv4, b2s, batch3)
    return (fused_graph, fused_node)


# speedup vs baseline: 42.9387x; 1.0506x over previous
import functools
import jax
import jax.numpy as jnp
from jax import lax
from jax.experimental import pallas as pl
from jax.experimental.pallas import tpu as pltpu
from jax.experimental.pallas import tpu_sc as plsc

N = 10000
E = 320000
G = 64
V = 16
D = 128
H = 256

NPAD = 10240
NPT = NPAD // 16
EROWS = 2560
RPT = EROWS // 16
ECH = 16
NCH = RPT // ECH
EPAD = EROWS * 128 - E
CFLAT = NPAD * V

_sc_kernel_cache = {}


def _sc_mesh():
    return plsc.VectorSubcoreMesh(core_axis_name="c", subcore_axis_name="s",
                                  num_cores=2, num_subcores=16)


def _rsqrt_sc(d):
    i = lax.bitcast_convert_type(d, jnp.int32)
    i = jnp.int32(0x5F3759DF) - lax.shift_right_logical(i, 1)
    y = lax.bitcast_convert_type(i, jnp.float32)
    for _ in range(3):
        y = y * (1.5 - 0.5 * d * y * y)
    return y


def _sc_deg_c(edges, edges_flat, x_p):
    if "deg_c" not in _sc_kernel_cache:
        _sc_kernel_cache["deg_c"] = functools.partial(
            pl.kernel,
            out_type=(
                jax.ShapeDtypeStruct((2, CFLAT), jnp.float32),
                jax.ShapeDtypeStruct((2, NPAD), jnp.float32),
            ),
            mesh=_sc_mesh(),
            compiler_params=pltpu.CompilerParams(needs_layout_passes=False),
            scratch_types=[
                pltpu.MemorySpace.VMEM_SHARED((CFLAT,), jnp.float32),
                pltpu.MemorySpace.VMEM_SHARED((NPAD,), jnp.float32),
                pltpu.MemorySpace.VMEM_SHARED((NPAD,), jnp.float32),
                pltpu.VMEM((CFLAT // 16,), jnp.float32),
                pltpu.VMEM((NPAD,), jnp.int32),
                pltpu.VMEM((NPAD,), jnp.float32),
                pltpu.VMEM((NPT,), jnp.float32),
                pltpu.VMEM((NPT,), jnp.float32),
                pltpu.VMEM((RPT, 128), jnp.int32),
                pltpu.VMEM((RPT * 128,), jnp.int32),
                pltpu.VMEM((RPT * 128,), jnp.int32),
                pltpu.VMEM((128,), jnp.int32),
                pltpu.VMEM((128,), jnp.float32),
                pltpu.VMEM((128,), jnp.int32),
                pltpu.VMEM((128,), jnp.float32),
                pltpu.VMEM((128,), jnp.float32),
                pltpu.SemaphoreType.DMA,
            ],
        )(_sc_deg_c_body)
    return _sc_kernel_cache["deg_c"](edges, edges_flat, x_p)


def _sc_deg_c_body(edges, edges_flat, x_hbm, c_out, dinv_out,
                   c_spm, deg_spm, dinv_spm,
                   zero_v, x_v, dinv_v, deg_sl, dinv_sl,
                   edgd, srcf, dstf, idx_a, upd_a, idx_b, upd_b, ones_v, sems):
    c = lax.axis_index("c")
    s = lax.axis_index("s")
    nz = CFLAT // 16

    def fill_zero(i, _):
        zero_v[pl.ds(i * 16, 16)] = jnp.zeros((16,), jnp.float32)
        return 0
    lax.fori_loop(0, nz // 16, fill_zero, 0)
    for i in range(8):
        ones_v[pl.ds(i * 16, 16)] = jnp.ones((16,), jnp.float32)

    pltpu.sync_copy(zero_v, c_spm.at[pl.ds(s * nz, nz)])
    pltpu.sync_copy(zero_v.at[pl.ds(0, NPT)], deg_spm.at[pl.ds(s * NPT, NPT)])

    pltpu.sync_copy(edges.at[c, 1, s], edgd)
    pltpu.sync_copy(edges_flat.at[c, 0, s], srcf)
    pltpu.sync_copy(edges_flat.at[c, 1, s], dstf)
    pltpu.sync_copy(x_hbm, x_v)
    plsc.subcore_barrier()

    def deg_row(j, _):
        @pl.when(j >= 8)
        def _():
            pltpu.make_async_copy(ones_v, deg_spm.at[edgd.at[j]], sems).wait()
        pltpu.async_copy(ones_v, deg_spm.at[edgd.at[j]], sems, add=True)
        return 0
    lax.fori_loop(0, RPT, deg_row, 0)

    def deg_drain(j, _):
        pltpu.make_async_copy(ones_v, deg_spm.at[edgd.at[j]], sems).wait()
        return 0
    lax.fori_loop(0, 8, deg_drain, 0)
    plsc.subcore_barrier()

    pltpu.sync_copy(deg_spm.at[pl.ds(s * NPT, NPT)], deg_sl)

    def dinv_blk(i, _):
        d = deg_sl[pl.ds(i * 16, 16)] + 1.0
        dinv_sl[pl.ds(i * 16, 16)] = _rsqrt_sc(d)
        return 0
    lax.fori_loop(0, NPT // 16, dinv_blk, 0)
    pltpu.sync_copy(dinv_sl, dinv_spm.at[pl.ds(s * NPT, NPT)])
    pltpu.sync_copy(dinv_sl, dinv_out.at[c, pl.ds(s * NPT, NPT)])
    plsc.subcore_barrier()

    pltpu.sync_copy(dinv_spm, dinv_v)

    def _c_build(j, idx_v, upd_v):
        for i in range(8):
            sl = pl.ds(i * 16, 16)
            fl = pl.ds(j * 128 + i * 16, 16)
            sv = srcf[fl]
            dv = dstf[fl]
            xv = plsc.load_gather(x_v, [sv])
            wv = plsc.load_gather(dinv_v, [sv])
            idx_v[sl] = dv * 16 + xv
            upd_v[sl] = wv
        pltpu.async_copy(upd_v, c_spm.at[idx_v], sems, add=True)

    def c_row(j, _):
        @pl.when(lax.rem(j, 2) == 0)
        def _():
            @pl.when(j >= 2)
            def _():
                pltpu.make_async_copy(upd_a, c_spm.at[idx_a], sems).wait()
            _c_build(j, idx_a, upd_a)

        @pl.when(lax.rem(j, 2) == 1)
        def _():
            @pl.when(j >= 2)
            def _():
                pltpu.make_async_copy(upd_b, c_spm.at[idx_b], sems).wait()
            _c_build(j, idx_b, upd_b)
        return 0
    lax.fori_loop(0, RPT, c_row, 0)
    pltpu.make_async_copy(upd_a, c_spm.at[idx_a], sems).wait()
    pltpu.make_async_copy(upd_b, c_spm.at[idx_b], sems).wait()
    plsc.subcore_barrier()

    pltpu.sync_copy(c_spm.at[pl.ds(s * nz, nz)], c_out.at[c, pl.ds(s * nz, nz)])


def _sc_aggregate(edges, t):
    if "agg" not in _sc_kernel_cache:
        _sc_kernel_cache["agg"] = functools.partial(
            pl.kernel,
            out_type=jax.ShapeDtypeStruct((2, NPAD, D), jnp.float32),
            mesh=_sc_mesh(),
            compiler_params=pltpu.CompilerParams(needs_layout_passes=False),
            scratch_types=[
                pltpu.MemorySpace.VMEM_SHARED((NPAD, D), jnp.float32),
                pltpu.VMEM((2, 2, ECH, 128), jnp.int32),
                pltpu.VMEM((2, 128, D), jnp.float32),
                pltpu.SemaphoreType.DMA,
                pltpu.SemaphoreType.DMA,
                pltpu.SemaphoreType.DMA,
            ],
        )(_sc_aggregate_body)
    return _sc_kernel_cache["agg"](edges, t)


def _sc_aggregate_body(edges, t_hbm, acc_out, acc_spm, edg, rows,
                       seme, semg, sems):
    c = lax.axis_index("c")
    s = lax.axis_index("s")

    nrp = NPAD // 16
    pltpu.sync_copy(t_hbm.at[c, pl.ds(s * nrp, nrp)],
                    acc_spm.at[pl.ds(s * nrp, nrp)])
    plsc.subcore_barrier()

    def echunk(k, km):
        return pltpu.make_async_copy(
            edges.at[c, :, s, pl.ds(k * ECH, ECH)], edg.at[km], seme)

    def gat(j, km, jj, jm):
        return pltpu.make_async_copy(
            t_hbm.at[c].at[edg.at[km, 0, jj]], rows.at[jm], semg)

    def sca(j, km, jj, jm):
        return pltpu.make_async_copy(
            rows.at[jm], acc_spm.at[edg.at[km, 1, jj]], sems)

    echunk(0, 0).start()
    echunk(0, 0).wait()

    @pl.when(NCH > 1)
    def _():
        echunk(1, 1).start()
    pltpu.async_copy(t_hbm.at[c].at[edg.at[0, 0, 0]], rows.at[0], semg)

    def agg_row(j, _):
        km = lax.rem(j // ECH, 2)
        jj = lax.rem(j, ECH)
        jm = lax.rem(j, 2)
        gat(j, km, jj, jm).wait()
        pltpu.async_copy(rows.at[jm], acc_spm.at[edg.at[km, 1, jj]], sems,
                         add=True)

        @pl.when(j >= 1)
        def _():
            sca(j, km, jj, 1 - jm).wait()

        @pl.when((jj == 0) & (j >= ECH) & (j + ECH < RPT))
        def _():
            echunk(j // ECH + 1, lax.rem(j // ECH + 1, 2)).start()

        @pl.when(jnp.logical_and(jj == ECH - 1, j + 1 < RPT))
        def _():
            echunk((j + 1) // ECH, lax.rem((j + 1) // ECH, 2)).wait()

        @pl.when(j + 1 < RPT)
        def _():
            km2 = lax.rem((j + 1) // ECH, 2)
            jj2 = lax.rem(j + 1, ECH)
            pltpu.async_copy(
                t_hbm.at[c].at[edg.at[km2, 0, jj2]], rows.at[1 - jm], semg)
        return 0
    lax.fori_loop(0, RPT, agg_row, 0)
    sca(RPT - 1, 0, 0, lax.rem(RPT - 1, 2)).wait()
    plsc.subcore_barrier()

    nout = NPAD // 16
    pltpu.sync_copy(acc_spm.at[pl.ds(s * nout, nout)],
                    acc_out.at[c, pl.ds(s * nout, nout)])


def _tc1_body(c_ref, dinv_ref, x_ref, emb_ref, w1_ref, b1_ref, w2_ref, t_ref):
    cb = c_ref[0]
    dv = dinv_ref[0, 0, 0]
    xv = x_ref[0, 0]
    bn = cb.shape[0]
    emb_w1 = jnp.dot(emb_ref[...], w1_ref[0], preferred_element_type=jnp.float32)
    oh = (lax.broadcasted_iota(jnp.int32, (bn, V), 1) == xv[:, None])
    ctot = cb * dv[:, None] + oh.astype(jnp.float32) * (dv * dv)[:, None]
    z1 = jnp.dot(ctot, emb_w1,
                 preferred_element_type=jnp.float32) + b1_ref[0, 0][None, :]
    h1 = jnp.where(z1 > 0, z1, jnp.exp(z1) - 1.0)
    t = jnp.dot(h1, w2_ref[0], preferred_element_type=jnp.float32) * dv[:, None]
    t_ref[0] = t


def _tc1(c3, dinv4, x3, emb, w1s, b1s, w2s):
    bn = 1024
    nb = NPAD // bn
    return pl.pallas_call(
        _tc1_body,
        grid=(2, nb),
        in_specs=[
            pl.BlockSpec((1, bn, V), lambda v, b: (v, b, 0)),
            pl.BlockSpec((1, 1, 1, bn), lambda v, b: (v, b, 0, 0)),
            pl.BlockSpec((1, 1, bn), lambda v, b: (b, 0, 0)),
            pl.BlockSpec((V, D), lambda v, b: (0, 0)),
            pl.BlockSpec((1, D, H), lambda v, b: (v, 0, 0)),
            pl.BlockSpec((1, 1, H), lambda v, b: (v, 0, 0)),
            pl.BlockSpec((1, H, D), lambda v, b: (v, 0, 0)),
        ],
        out_specs=pl.BlockSpec((1, bn, D), lambda v, b: (v, b, 0)),
        out_shape=jax.ShapeDtypeStruct((2, NPAD, D), jnp.float32),
    )(c3, dinv4, x3, emb, w1s, b1s, w2s)


def _tc2_body(acc_ref, dinv_ref, b2_ref, batch_ref, node_ref, graph_ref,
              gacc, gcnt):
    b = pl.program_id(0)
    nb = pl.num_programs(0)
    bn = acc_ref.shape[1]

    @pl.when(b == 0)
    def _():
        gacc[...] = jnp.zeros_like(gacc)
        gcnt[...] = jnp.zeros_like(gcnt)

    hs = []
    for v in range(2):
        z = acc_ref[v] * dinv_ref[v, 0, 0][:, None] + b2_ref[v][None, :]
        hs.append(jnp.where(z > 0, z, jnp.exp(z) - 1.0))
    fused = jnp.concatenate(hs, axis=1)
    node_ref[...] = fused

    row = b * bn + lax.broadcasted_iota(jnp.int32, (1, bn), 1)[0]
    valid = row < N
    bv = batch_ref[0, 0]
    pt = (lax.broadcasted_iota(jnp.int32, (G, bn), 0) == bv[None, :]) \
        & valid[None, :]
    ptf = pt.astype(jnp.float32)
    g_new = gacc[...] + jnp.dot(ptf, fused, preferred_element_type=jnp.float32)
    c_new = gcnt[...] + jnp.dot(ptf, jnp.ones((bn, D), jnp.float32),
                                preferred_element_type=jnp.float32)
    gacc[...] = g_new
    gcnt[...] = c_new

    @pl.when(b == nb - 1)
    def _():
        graph_ref[...] = g_new / jnp.maximum(c_new[:, 0:1], 1.0)


def _tc2(acc, dinv4, b2s, batch3):
    bn = 1024
    nb = NPAD // bn
    return pl.pallas_call(
        _tc2_body,
        grid=(nb,),
        in_specs=[
            pl.BlockSpec((2, bn, D), lambda b: (0, b, 0)),
            pl.BlockSpec((2, 1, 1, bn), lambda b: (0, b, 0, 0)),
            pl.BlockSpec((2, D), lambda b: (0, 0)),
            pl.BlockSpec((1, 1, bn), lambda b: (b, 0, 0)),
        ],
        out_specs=[
            pl.BlockSpec((bn, 2 * D), lambda b: (b, 0)),
            pl.BlockSpec((G, 2 * D), lambda b: (0, 0)),
        ],
        out_shape=[
            jax.ShapeDtypeStruct((N, 2 * D), jnp.float32),
            jax.ShapeDtypeStruct((G, 2 * D), jnp.float32),
        ],
        scratch_shapes=[
            pltpu.VMEM((G, 2 * D), jnp.float32),
            pltpu.VMEM((G, D), jnp.float32),
        ],
    )(acc, dinv4, b2s, batch3)


def kernel(x, edge_index_seq, edge_index_pair, batch, emb,
           Ws1, bs1, Ws2, bs2, Wp1, bp1, Wp2, bp2):
    i32 = jnp.int32
    ei = jnp.stack([edge_index_seq.astype(i32), edge_index_pair.astype(i32)])
    pad_src = jnp.broadcast_to((jnp.arange(EPAD, dtype=i32) % 128)[None, :],
                               (2, EPAD))
    pad_dst = jnp.full((2, EPAD), N, i32)
    src = jnp.concatenate([ei[:, 0, :], pad_src], axis=1)
    dst = jnp.concatenate([ei[:, 1, :], pad_dst], axis=1)
    edges = jnp.stack([src, dst], axis=1).reshape(2, 2, 16, RPT, 128)
    edges_flat = edges.reshape(2, 2, 16, RPT * 128)

    x_p = jnp.concatenate([x.astype(i32), jnp.zeros((NPAD - N,), i32)])
    batch_p = jnp.concatenate([batch.astype(i32), jnp.zeros((NPAD - N,), i32)])

    w1s = jnp.stack([Ws1, Wp1])
    b1s = jnp.stack([bs1, bp1]).reshape(2, 1, H)
    w2s = jnp.stack([Ws2, Wp2])
    b2s = jnp.stack([bs2, bp2])

    c_flat, dinv = _sc_deg_c(edges, edges_flat, x_p)
    c3 = c_flat.reshape(2, NPAD, V)
    dinv4 = dinv.reshape(2, NPAD // 1024, 1, 1024)
    x3 = x_p.reshape(NPAD // 1024, 1, 1024)
    batch3 = batch_p.reshape(NPAD // 1024, 1, 1024)

    t = _tc1(c3, dinv4, x3, emb, w1s, b1s, w2s)
    acc = _sc_aggregate(edges, t)
    fused_node, fused_graph = _tc2(acc, dinv4, b2s, batch3)
    return (fused_graph, fused_node)
